# baseline (speedup 1.0000x reference)
import functools

import jax
import jax.numpy as jnp
from jax import lax
from jax.experimental import pallas as pl
from jax.experimental.pallas import tpu as pltpu
from jax.experimental.pallas import tpu_sc as plsc

_NC = 2
_NS = 16
_L = 16


def _sc_body(b, d, h, w, col_hbm, row_hbm, out_hbm, tbl_v, pat1, pat2, sem):
    d2 = d // 2
    sid = lax.axis_index("s")
    wid = sid * _NC + lax.axis_index("c")

    pltpu.sync_copy(col_hbm, tbl_v.at[pl.ds(0, w * d2)])
    pltpu.sync_copy(row_hbm, tbl_v.at[pl.ds(w * d2, h * d2)])

    rbase = w * d2 + wid * d2
    rvecs = [tbl_v[pl.ds(rbase + c * _L, _L)] for c in range(d2 // _L)]

    for ww in range(w):
        for c in range(d2 // _L):
            pat1[pl.ds(ww * d + c * _L, _L)] = tbl_v[pl.ds(ww * d2 + c * _L, _L)]
            pat1[pl.ds(ww * d + d2 + c * _L, _L)] = rvecs[c]

    local = [
        pltpu.make_async_copy(pat1.at[pl.ds(ww * d, d)], pat2.at[sid, ww], sem)
        for ww in range(w)
    ]
    for c in local:
        c.start()
    for c in local:
        c.wait()

    copies = [
        pltpu.make_async_copy(pat2.at[sid], out_hbm.at[bi, wid], sem)
        for bi in range(b)
    ]
    for c in copies:
        c.start()
    for c in copies:
        c.wait()


def kernel(tensor, row_embed, col_embed):
    b = tensor.shape[0]
    h, w = tensor.shape[-2], tensor.shape[-1]
    d2 = row_embed.shape[-1]
    d = 2 * d2
    mesh = plsc.VectorSubcoreMesh(core_axis_name="c", subcore_axis_name="s")
    sc_call = pl.kernel(
        functools.partial(_sc_body, b, d, h, w),
        out_type=jax.ShapeDtypeStruct((b, h, w, d), jnp.float32),
        mesh=mesh,
        scratch_types=[
            pltpu.VMEM(((w + h) * d2,), jnp.float32),
            pltpu.VMEM((w * d,), jnp.float32),
            pltpu.VMEM_SHARED((_NS, w, d), jnp.float32),
            pltpu.SemaphoreType.DMA,
        ],
    )
    out = sc_call(col_embed[:w].reshape(-1), row_embed[:h].reshape(-1))
    return jnp.transpose(out, (0, 3, 1, 2))


# speedup vs baseline: 4.9641x; 4.9641x over previous
import jax
import jax.numpy as jnp
from jax.experimental import pallas as pl
from jax.experimental.pallas import tpu as pltpu


def _body(col_ref, row_ref, out_ref, pat_ref):
    w, d2 = col_ref.shape
    h = row_ref.shape[0]

    @pl.when(pl.program_id(0) == 0)
    def _():
        x_part = jnp.broadcast_to(col_ref[...][None, :, :], (h, w, d2))
        y_part = jnp.broadcast_to(row_ref[...][:, None, :], (h, w, d2))
        pat_ref[...] = jnp.concatenate([x_part, y_part], axis=-1)

    for j in range(out_ref.shape[0]):
        out_ref[j] = pat_ref[...]


def kernel(tensor, row_embed, col_embed):
    b = tensor.shape[0]
    h, w = tensor.shape[-2], tensor.shape[-1]
    d2 = row_embed.shape[-1]
    d = 2 * d2
    out = pl.pallas_call(
        _body,
        grid=(b // 4,),
        in_specs=[
            pl.BlockSpec((w, d2), lambda i: (0, 0)),
            pl.BlockSpec((h, d2), lambda i: (0, 0)),
        ],
        out_specs=pl.BlockSpec((4, h, w, d), lambda i: (i, 0, 0, 0)),
        out_shape=jax.ShapeDtypeStruct((b, h, w, d), jnp.float32),
        scratch_shapes=[
            pltpu.VMEM((h, w, d), jnp.float32),
        ],
    )(col_embed, row_embed)
    return jnp.transpose(out, (0, 3, 1, 2))
